# baseline (speedup 1.0000x reference)
import jax
import jax.numpy as jnp
from jax import lax
from jax.experimental import pallas as pl
from jax.experimental.pallas import tpu as pltpu
from jax.experimental.pallas import tpu_sc as plsc

_NC, _NS = 2, 16
_F32 = jnp.float32
_HIGH = lax.Precision.HIGHEST


def _mesh():
    return plsc.VectorSubcoreMesh(core_axis_name="c", subcore_axis_name="s",
                                  num_cores=_NC, num_subcores=_NS)


_SC_PARAMS = pltpu.CompilerParams(use_tc_tiling_on_sc=False)


def _cdiv(a, b):
    return (a + b - 1) // b


def _pad_edges(ei):
    e = ei.shape[1]
    ep = _cdiv(e, 32768) * 32768
    pad = ep - e
    src = jnp.concatenate([ei[0], jnp.zeros((pad,), jnp.int32)])
    dst = jnp.concatenate([ei[1], jnp.full((pad,), -1, jnp.int32)])
    return src.reshape(-1, 128), dst.reshape(-1, 128)


def _sc_counts(dsts, n_dsts, core_of):
    nps = [_cdiv(n + 1, 128) * 128 for n in n_dsts]
    np_max = max(nps)
    outs = [jax.ShapeDtypeStruct((npr, 16), _F32) for npr in nps]
    nrel = len(dsts)

    def body(*refs):
        z_hbm, ones_hbm = refs[0], refs[1]
        d_hbms = refs[2:2 + nrel]
        o_hbms = refs[2 + nrel:2 + 2 * nrel]
        cnt, zc, ones_v, idx_d = refs[2 + 2 * nrel:]
        c = lax.axis_index("c")
        t = lax.axis_index("s")
        pltpu.sync_copy(z_hbm, zc)
        pltpu.sync_copy(ones_hbm, ones_v)
        for r in range(nrel):
            d_hbm, o_hbm, n, npr = d_hbms[r], o_hbms[r], n_dsts[r], nps[r]

            @pl.when(c == core_of[r])
            def _():
                rows_t = npr // _NS
                r0 = t * rows_t
                nzf, nzr = rows_t // 1024, rows_t % 1024
                for j in range(nzf):
                    pltpu.sync_copy(zc, cnt.at[pl.ds(r0 + j * 1024, 1024), :])
                if nzr:
                    pltpu.sync_copy(zc.at[pl.ds(0, nzr), :],
                                    cnt.at[pl.ds(r0 + nzf * 1024, nzr), :])
                plsc.subcore_barrier()
                erows = d_hbm.shape[0]
                erows_t = erows // _NS
                rbase = t * erows_t

                @pl.loop(0, erows_t)
                def _(b):
                    pltpu.sync_copy(d_hbm.at[pl.ds(rbase + b, 1), :], idx_d)
                    for g in range(8):
                        sl = pl.ds(g * 16, 16)
                        dv = idx_d[0, sl]
                        idx_d[0, sl] = jnp.where(dv >= 0, dv, n)
                    pltpu.sync_copy(ones_v, cnt.at[idx_d.at[0]], add=True)

                plsc.subcore_barrier()
                pltpu.sync_copy(cnt.at[pl.ds(r0, rows_t), :],
                                o_hbm.at[pl.ds(r0, rows_t), :])
                plsc.subcore_barrier()

    kern = pl.kernel(
        body, out_type=outs, mesh=_mesh(), compiler_params=_SC_PARAMS,
        scratch_types=[
            pltpu.VMEM_SHARED((np_max, 16), _F32),
            pltpu.VMEM((1024, 16), _F32),
            pltpu.VMEM((128, 16), _F32),
            pltpu.VMEM((1, 128), jnp.int32),
        ])
    zeros = jnp.zeros((1024, 16), _F32)
    ones = jnp.zeros((128, 16), _F32).at[:, 0].set(1.0)
    res = kern(zeros, ones, *dsts)
    return tuple(res) if isinstance(res, (list, tuple)) else (res,)


def _sc_agg(n_dst, rel_specs):
    c_rows = n_dst // 2
    cp = _cdiv(c_rows + 1, 128) * 128
    rows_t = cp // _NS
    nzf, nzr = rows_t // 128, rows_t % 128
    nrel = len(rel_specs)
    outs = [jax.ShapeDtypeStruct((2, 4, cp, 32), _F32) for _ in rel_specs]

    def body(*refs):
        z_hbm = refs[0]
        rels = refs[1:1 + 3 * nrel]
        o_hbms = refs[1 + 3 * nrel:1 + 4 * nrel]
        scr = refs[1 + 4 * nrel:]
        acc = scr[0]
        idx_s = [scr[1 + b] for b in range(4)]
        idx_d = [scr[5 + b] for b in range(4)]
        rowbuf = [[scr[9 + 4 * b + j] for j in range(4)] for b in range(4)]
        sem_i, sem_g, sem_s = scr[25], scr[26], scr[27]
        c = lax.axis_index("c")
        t = lax.axis_index("s")
        lo = c * c_rows
        r0 = t * rows_t
        for r in range(nrel):
            y4, s_hbm, d_hbm = rels[3 * r:3 * r + 3]
            erows = s_hbm.shape[0]
            erows_t = erows // _NS
            nsb = erows_t // 4
            rbase = t * erows_t

            @pl.loop(0, 4)
            def _(p):
                yh = y4.at[p]
                for j in range(nzf):
                    pltpu.sync_copy(z_hbm, acc.at[pl.ds(r0 + j * 128, 128), :])
                if nzr:
                    pltpu.sync_copy(z_hbm.at[pl.ds(0, nzr), :],
                                    acc.at[pl.ds(r0 + nzf * 128, nzr), :])
                plsc.subcore_barrier()

                pltpu.async_copy(s_hbm.at[pl.ds(rbase, 4), :], idx_s[0],
                                 sem_i.at[0])
                pltpu.async_copy(d_hbm.at[pl.ds(rbase, 4), :], idx_d[0],
                                 sem_i.at[0])

                def run_sb(g, b):
                    bn = (b + 1) % 4
                    sl = pl.ds(rbase + g * 4, 4)
                    pltpu.make_async_copy(s_hbm.at[sl, :], idx_s[b],
                                          sem_i.at[b]).wait()
                    pltpu.make_async_copy(d_hbm.at[sl, :], idx_d[b],
                                          sem_i.at[b]).wait()

                    @pl.loop(0, 4)
                    def _(i):
                        @pl.loop(0, 8)
                        def _(q):
                            slq = pl.ds(q * 16, 16)
                            dv = idx_d[b][i, slq]
                            ok = (dv >= lo) & (dv < lo + c_rows)
                            idx_d[b][i, slq] = jnp.where(ok, dv - lo, c_rows)

                    gds = [pltpu.async_copy(yh.at[idx_s[b].at[j]],
                                            rowbuf[b][j], sem_g.at[j])
                           for j in range(4)]

                    @pl.when((g >= 3) & (g < nsb - 1))
                    def _():
                        for j in range(4):
                            pltpu.make_async_copy(
                                rowbuf[bn][j], acc.at[idx_d[bn].at[j]],
                                sem_s.at[bn]).wait()

                    @pl.when(g < nsb - 1)
                    def _():
                        sl2 = pl.ds(rbase + (g + 1) * 4, 4)
                        pltpu.async_copy(s_hbm.at[sl2, :], idx_s[bn],
                                         sem_i.at[bn])
                        pltpu.async_copy(d_hbm.at[sl2, :], idx_d[bn],
                                         sem_i.at[bn])

                    for j in range(4):
                        gds[j].wait()
                        pltpu.async_copy(rowbuf[b][j],
                                         acc.at[idx_d[b].at[j]],
                                         sem_s.at[b], add=True)

                @pl.loop(0, nsb // 4)
                def _(g4):
                    for k in range(4):
                        run_sb(g4 * 4 + k, k)

                for bb in range(4):
                    for j in range(4):
                        pltpu.make_async_copy(rowbuf[bb][j],
                                              acc.at[idx_d[bb].at[j]],
                                              sem_s.at[bb]).wait()
                plsc.subcore_barrier()
                pltpu.sync_copy(acc.at[pl.ds(r0, rows_t), :],
                                o_hbms[r].at[c, p, pl.ds(r0, rows_t), :])
                plsc.subcore_barrier()

    kern = pl.kernel(
        body, out_type=outs, mesh=_mesh(), compiler_params=_SC_PARAMS,
        scratch_types=(
            [pltpu.VMEM_SHARED((cp, 32), _F32)]
            + [pltpu.VMEM((4, 128), jnp.int32) for _ in range(8)]
            + [pltpu.VMEM((128, 32), _F32) for _ in range(16)]
            + [pltpu.SemaphoreType.DMA((4,)),
               pltpu.SemaphoreType.DMA((4,)),
               pltpu.SemaphoreType.DMA((4,))]))
    zeros = jnp.zeros((128, 32), _F32)
    flat = []
    for spec in rel_specs:
        flat.extend([spec[0], spec[1], spec[2]])
    res = kern(zeros, *flat)
    return tuple(res) if isinstance(res, (list, tuple)) else (res,)


def _assemble(agg, n_dst):
    c_rows = n_dst // 2
    top = jnp.concatenate([agg[0, q, :c_rows] for q in range(4)], axis=1)
    bot = jnp.concatenate([agg[1, q, :c_rows] for q in range(4)], axis=1)
    return jnp.concatenate([top, bot], axis=0)


def _mm_multi(x, w_cat, widths, br=1024):
    n, k = x.shape
    m = w_cat.shape[1]
    grid = _cdiv(n, br)

    def body(x_ref, w_ref, *o_refs):
        acc = jnp.dot(x_ref[...], w_ref[...],
                      preferred_element_type=_F32, precision=_HIGH)
        off = 0
        for o_ref, w in zip(o_refs, widths):
            o_ref[...] = acc[:, off:off + w]
            off += w

    return pl.pallas_call(
        body,
        grid=(grid,),
        in_specs=[pl.BlockSpec((br, k), lambda i: (i, 0)),
                  pl.BlockSpec((k, m), lambda i: (0, 0))],
        out_specs=[pl.BlockSpec((br, w), lambda i: (i, 0)) for w in widths],
        out_shape=[jax.ShapeDtypeStruct((n, w), _F32) for w in widths],
    )(x, w_cat)


def _combine(aggs, cnts, x, wr_stack, bl_stack, proj_w=None, proj_b=None,
             br=1024):
    n = x.shape[0]
    nrel = len(aggs)
    grid = _cdiv(n, br)
    m = proj_w.shape[1] if proj_w is not None else 128

    def body(*refs):
        a_refs = refs[:nrel]
        c_refs = refs[nrel:2 * nrel]
        x_ref = refs[2 * nrel]
        wr_ref = refs[2 * nrel + 1]
        bl_ref = refs[2 * nrel + 2]
        o_ref = refs[-1]
        wsum = jnp.sum(wr_ref[...], axis=0)
        acc = jnp.dot(x_ref[...], wsum,
                      preferred_element_type=_F32, precision=_HIGH)
        acc = acc + jnp.sum(bl_ref[...], axis=0)[None, :]
        for a_ref, c_ref in zip(a_refs, c_refs):
            acc = acc + a_ref[...] * (1.0 / jnp.maximum(c_ref[...], 1.0))
        h = jnp.maximum(acc, 0.0)
        if proj_w is not None:
            pw_ref, pb_ref = refs[2 * nrel + 3], refs[2 * nrel + 4]
            o_ref[...] = jnp.dot(h, pw_ref[...], preferred_element_type=_F32,
                                 precision=_HIGH) + pb_ref[...]
        else:
            o_ref[...] = h

    in_specs = (
        [pl.BlockSpec((br, 128), lambda i: (i, 0)) for _ in range(nrel)]
        + [pl.BlockSpec((br, 1), lambda i: (i, 0)) for _ in range(nrel)]
        + [pl.BlockSpec((br, 128), lambda i: (i, 0)),
           pl.BlockSpec((nrel, 128, 128), lambda i: (0, 0, 0)),
           pl.BlockSpec((nrel, 128), lambda i: (0, 0))])
    args = list(aggs) + list(cnts) + [x, wr_stack, bl_stack]
    if proj_w is not None:
        in_specs += [pl.BlockSpec((128, m), lambda i: (0, 0)),
                     pl.BlockSpec((1, m), lambda i: (0, 0))]
        args += [proj_w, proj_b]

    return pl.pallas_call(
        body,
        grid=(grid,),
        in_specs=in_specs,
        out_specs=pl.BlockSpec((br, m), lambda i: (i, 0)),
        out_shape=jax.ShapeDtypeStruct((n, m), _F32),
    )(*args)


def kernel(x_paper, x_author, x_institution, x_fos, ei_cites, ei_writes,
           ei_rev_writes, ei_aff, ei_rev_aff, ei_topic, ei_rev_topic,
           W_l, b_l, W_r, W_out, b_out):
    n_p, n_a, n_f = x_paper.shape[0], x_author.shape[0], x_fos.shape[0]
    out_dim = W_out.shape[1]

    s0, d0 = _pad_edges(ei_cites)
    s1, d1 = _pad_edges(ei_writes)
    s2, d2 = _pad_edges(ei_rev_writes)
    s4, d4 = _pad_edges(ei_rev_aff)
    s5, d5 = _pad_edges(ei_topic)
    s6, d6 = _pad_edges(ei_rev_topic)

    c0, c1, c2, c4, c5, c6 = _sc_counts(
        [d0, d1, d2, d4, d5, d6],
        [n_p, n_p, n_a, n_a, n_f, n_p],
        core_of=[0, 0, 1, 1, 1, 1])
    c0, c1, c6 = c0[:n_p, 0:1], c1[:n_p, 0:1], c6[:n_p, 0:1]
    c2, c4, c5 = c2[:n_a, 0:1], c4[:n_a, 0:1], c5[:n_f, 0:1]

    def to4(y):
        return jnp.transpose(y.reshape(y.shape[0], 4, 32), (1, 0, 2))

    y0, y2, y5 = _mm_multi(
        x_paper, jnp.concatenate([W_l[0, 0], W_l[0, 2], W_l[0, 5]], axis=1),
        [128] * 3)
    y0, y2, y5 = to4(y0), to4(y2), to4(y5)
    (y1,) = _mm_multi(x_author, W_l[0, 1], [128])
    (y4,) = _mm_multi(x_institution, W_l[0, 4], [128])
    (y6,) = _mm_multi(x_fos, W_l[0, 6], [128])
    y1, y4, y6 = to4(y1), to4(y4), to4(y6)

    a0, a1, a6 = _sc_agg(n_p, [(y0, s0, d0), (y1, s1, d1), (y6, s6, d6)])
    a2, a4 = _sc_agg(n_a, [(y2, s2, d2), (y4, s4, d4)])
    (a5,) = _sc_agg(n_f, [(y5, s5, d5)])

    h_p = _combine([_assemble(a0, n_p), _assemble(a1, n_p),
                    _assemble(a6, n_p)],
                   [c0, c1, c6], x_paper,
                   jnp.stack([W_r[0, 0], W_r[0, 1], W_r[0, 6]]),
                   jnp.stack([b_l[0, 0], b_l[0, 1], b_l[0, 6]]))
    h_a = _combine([_assemble(a2, n_a), _assemble(a4, n_a)],
                   [c2, c4], x_author,
                   jnp.stack([W_r[0, 2], W_r[0, 4]]),
                   jnp.stack([b_l[0, 2], b_l[0, 4]]))
    h_f = _combine([_assemble(a5, n_f)], [c5], x_fos,
                   jnp.stack([W_r[0, 5]]), jnp.stack([b_l[0, 5]]))

    (z0,) = _mm_multi(h_p, W_l[1, 0], [128])
    (z1,) = _mm_multi(h_a, W_l[1, 1], [128])
    (z6,) = _mm_multi(h_f, W_l[1, 6], [128])
    g0, g1, g6 = _sc_agg(n_p, [(to4(z0), s0, d0), (to4(z1), s1, d1),
                               (to4(z6), s6, d6)])

    m_pad = _cdiv(out_dim, 128) * 128
    w_out_p = jnp.pad(W_out, ((0, 0), (0, m_pad - out_dim)))
    b_out_p = jnp.pad(b_out, (0, m_pad - out_dim))[None, :]
    logits = _combine(
        [_assemble(g0, n_p), _assemble(g1, n_p), _assemble(g6, n_p)],
        [c0, c1, c6], h_p,
        jnp.stack([W_r[1, 0], W_r[1, 1], W_r[1, 6]]),
        jnp.stack([b_l[1, 0], b_l[1, 1], b_l[1, 6]]),
        proj_w=w_out_p, proj_b=b_out_p)
    return logits[:, :out_dim]


# speedup vs baseline: 2.0846x; 2.0846x over previous
import jax
import jax.numpy as jnp
from jax import lax
from jax.experimental import pallas as pl
from jax.experimental.pallas import tpu as pltpu
from jax.experimental.pallas import tpu_sc as plsc

_NC, _NS = 2, 16
_F32 = jnp.float32
_HIGH = lax.Precision.HIGHEST


def _mesh():
    return plsc.VectorSubcoreMesh(core_axis_name="c", subcore_axis_name="s",
                                  num_cores=_NC, num_subcores=_NS)


_SC_PARAMS = pltpu.CompilerParams(use_tc_tiling_on_sc=False)


def _cdiv(a, b):
    return (a + b - 1) // b


def _pad_edges(ei):
    e = ei.shape[1]
    ep = _cdiv(e, 4096) * 4096
    pad = ep - e
    src = jnp.concatenate([ei[0], jnp.zeros((pad,), jnp.int32)])
    dst = jnp.concatenate([ei[1], jnp.full((pad,), -1, jnp.int32)])
    return src, dst


def _sc_counts(dsts, n_dsts, core_of):
    nps = [_cdiv(n + 1, 128) * 128 for n in n_dsts]
    np_max = max(nps)
    outs = [jax.ShapeDtypeStruct((npr, 16), _F32) for npr in nps]
    nrel = len(dsts)

    def body(*refs):
        z_hbm, ones_hbm = refs[0], refs[1]
        d_hbms = refs[2:2 + nrel]
        o_hbms = refs[2 + nrel:2 + 2 * nrel]
        cnt, zc, ones_v, idx_d = refs[2 + 2 * nrel:]
        c = lax.axis_index("c")
        t = lax.axis_index("s")
        pltpu.sync_copy(z_hbm, zc)
        pltpu.sync_copy(ones_hbm, ones_v)
        for r in range(nrel):
            d_hbm, o_hbm, n, npr = d_hbms[r], o_hbms[r], n_dsts[r], nps[r]

            @pl.when(c == core_of[r])
            def _():
                rows_t = npr // _NS
                r0 = t * rows_t
                nzf, nzr = rows_t // 1024, rows_t % 1024
                for j in range(nzf):
                    pltpu.sync_copy(zc, cnt.at[pl.ds(r0 + j * 1024, 1024), :])
                if nzr:
                    pltpu.sync_copy(zc.at[pl.ds(0, nzr), :],
                                    cnt.at[pl.ds(r0 + nzf * 1024, nzr), :])
                plsc.subcore_barrier()
                ep = d_hbm.shape[0]
                es = ep // _NS
                ebase = t * es

                @pl.loop(0, es // 128)
                def _(b):
                    pltpu.sync_copy(d_hbm.at[pl.ds(ebase + b * 128, 128)],
                                    idx_d)
                    for g in range(8):
                        sl = pl.ds(g * 16, 16)
                        dv = idx_d[sl]
                        idx_d[sl] = jnp.where(dv >= 0, dv, n)
                    pltpu.sync_copy(ones_v, cnt.at[idx_d], add=True)

                plsc.subcore_barrier()
                pltpu.sync_copy(cnt.at[pl.ds(r0, rows_t), :],
                                o_hbm.at[pl.ds(r0, rows_t), :])
                plsc.subcore_barrier()

    kern = pl.kernel(
        body, out_type=outs, mesh=_mesh(), compiler_params=_SC_PARAMS,
        scratch_types=[
            pltpu.VMEM_SHARED((np_max, 16), _F32),
            pltpu.VMEM((1024, 16), _F32),
            pltpu.VMEM((128, 16), _F32),
            pltpu.VMEM((128,), jnp.int32),
        ])
    zeros = jnp.zeros((1024, 16), _F32)
    ones = jnp.zeros((128, 16), _F32).at[:, 0].set(1.0)
    res = kern(zeros, ones, *dsts)
    return tuple(res) if isinstance(res, (list, tuple)) else (res,)


def _sc_agg(n_dst, rel_specs):
    c_rows = n_dst // 2
    cp = _cdiv(c_rows + 1, 128) * 128
    rows_t = cp // _NS
    nzf, nzr = rows_t // 128, rows_t % 128
    nrel = len(rel_specs)
    outs = [jax.ShapeDtypeStruct((2, 2, cp, 64), _F32) for _ in rel_specs]

    def body(*refs):
        z_hbm = refs[0]
        rels = refs[1:1 + 4 * nrel]
        o_hbms = refs[1 + 4 * nrel:1 + 5 * nrel]
        acc, zbuf, idx_s, idx_d, rows = refs[1 + 5 * nrel:]
        c = lax.axis_index("c")
        t = lax.axis_index("s")
        lo = c * c_rows
        r0 = t * rows_t
        pltpu.sync_copy(z_hbm, zbuf)
        for r in range(nrel):
            y0, y1, s_hbm, d_hbm = rels[4 * r:4 * r + 4]
            ep = s_hbm.shape[0]
            es = ep // _NS
            ebase = t * es
            for p in range(2):
                yh = y0 if p == 0 else y1
                for j in range(nzf):
                    pltpu.sync_copy(zbuf, acc.at[pl.ds(r0 + j * 128, 128), :])
                if nzr:
                    pltpu.sync_copy(zbuf.at[pl.ds(0, nzr), :],
                                    acc.at[pl.ds(r0 + nzf * 128, nzr), :])
                plsc.subcore_barrier()

                @pl.loop(0, es // 128)
                def _(b):
                    e0 = ebase + b * 128
                    pltpu.sync_copy(s_hbm.at[pl.ds(e0, 128)], idx_s)
                    pltpu.sync_copy(d_hbm.at[pl.ds(e0, 128)], idx_d)
                    for g in range(8):
                        sl = pl.ds(g * 16, 16)
                        dv = idx_d[sl]
                        ok = (dv >= lo) & (dv < lo + c_rows)
                        idx_d[sl] = jnp.where(ok, dv - lo, c_rows)
                    pltpu.sync_copy(yh.at[idx_s], rows)
                    pltpu.sync_copy(rows, acc.at[idx_d], add=True)

                plsc.subcore_barrier()
                pltpu.sync_copy(acc.at[pl.ds(r0, rows_t), :],
                                o_hbms[r].at[c, p, pl.ds(r0, rows_t), :])
                plsc.subcore_barrier()

    kern = pl.kernel(
        body, out_type=outs, mesh=_mesh(), compiler_params=_SC_PARAMS,
        scratch_types=[
            pltpu.VMEM_SHARED((cp, 64), _F32),
            pltpu.VMEM((128, 64), _F32),
            pltpu.VMEM((128,), jnp.int32),
            pltpu.VMEM((128,), jnp.int32),
            pltpu.VMEM((128, 64), _F32),
        ])
    zeros = jnp.zeros((128, 64), _F32)
    flat = [x for spec in rel_specs for x in spec]
    res = kern(zeros, *flat)
    return tuple(res) if isinstance(res, (list, tuple)) else (res,)


def _assemble(agg, n_dst):
    c_rows = n_dst // 2
    top = jnp.concatenate([agg[0, 0, :c_rows], agg[0, 1, :c_rows]], axis=1)
    bot = jnp.concatenate([agg[1, 0, :c_rows], agg[1, 1, :c_rows]], axis=1)
    return jnp.concatenate([top, bot], axis=0)


def _mm_multi(x, w_cat, widths, br=1024):
    n, k = x.shape
    m = w_cat.shape[1]
    grid = _cdiv(n, br)

    def body(x_ref, w_ref, *o_refs):
        acc = jnp.dot(x_ref[...], w_ref[...],
                      preferred_element_type=_F32, precision=_HIGH)
        off = 0
        for o_ref, w in zip(o_refs, widths):
            o_ref[...] = acc[:, off:off + w]
            off += w

    return pl.pallas_call(
        body,
        grid=(grid,),
        in_specs=[pl.BlockSpec((br, k), lambda i: (i, 0)),
                  pl.BlockSpec((k, m), lambda i: (0, 0))],
        out_specs=[pl.BlockSpec((br, w), lambda i: (i, 0)) for w in widths],
        out_shape=[jax.ShapeDtypeStruct((n, w), _F32) for w in widths],
    )(x, w_cat)


def _combine(aggs, cnts, x, wr_stack, bl_stack, proj_w=None, proj_b=None,
             br=1024):
    n = x.shape[0]
    nrel = len(aggs)
    grid = _cdiv(n, br)
    m = proj_w.shape[1] if proj_w is not None else 128

    def body(*refs):
        a_refs = refs[:nrel]
        c_refs = refs[nrel:2 * nrel]
        x_ref = refs[2 * nrel]
        wr_ref = refs[2 * nrel + 1]
        bl_ref = refs[2 * nrel + 2]
        o_ref = refs[-1]
        wsum = jnp.sum(wr_ref[...], axis=0)
        acc = jnp.dot(x_ref[...], wsum,
                      preferred_element_type=_F32, precision=_HIGH)
        acc = acc + jnp.sum(bl_ref[...], axis=0)[None, :]
        for a_ref, c_ref in zip(a_refs, c_refs):
            acc = acc + a_ref[...] * (1.0 / jnp.maximum(c_ref[...], 1.0))
        h = jnp.maximum(acc, 0.0)
        if proj_w is not None:
            pw_ref, pb_ref = refs[2 * nrel + 3], refs[2 * nrel + 4]
            o_ref[...] = jnp.dot(h, pw_ref[...], preferred_element_type=_F32,
                                 precision=_HIGH) + pb_ref[...]
        else:
            o_ref[...] = h

    in_specs = (
        [pl.BlockSpec((br, 128), lambda i: (i, 0)) for _ in range(nrel)]
        + [pl.BlockSpec((br, 1), lambda i: (i, 0)) for _ in range(nrel)]
        + [pl.BlockSpec((br, 128), lambda i: (i, 0)),
           pl.BlockSpec((nrel, 128, 128), lambda i: (0, 0, 0)),
           pl.BlockSpec((nrel, 128), lambda i: (0, 0))])
    args = list(aggs) + list(cnts) + [x, wr_stack, bl_stack]
    if proj_w is not None:
        in_specs += [pl.BlockSpec((128, m), lambda i: (0, 0)),
                     pl.BlockSpec((1, m), lambda i: (0, 0))]
        args += [proj_w, proj_b]

    return pl.pallas_call(
        body,
        grid=(grid,),
        in_specs=in_specs,
        out_specs=pl.BlockSpec((br, m), lambda i: (i, 0)),
        out_shape=jax.ShapeDtypeStruct((n, m), _F32),
    )(*args)


def kernel(x_paper, x_author, x_institution, x_fos, ei_cites, ei_writes,
           ei_rev_writes, ei_aff, ei_rev_aff, ei_topic, ei_rev_topic,
           W_l, b_l, W_r, W_out, b_out):
    n_p, n_a, n_f = x_paper.shape[0], x_author.shape[0], x_fos.shape[0]
    out_dim = W_out.shape[1]

    s0, d0 = _pad_edges(ei_cites)
    s1, d1 = _pad_edges(ei_writes)
    s2, d2 = _pad_edges(ei_rev_writes)
    s4, d4 = _pad_edges(ei_rev_aff)
    s5, d5 = _pad_edges(ei_topic)
    s6, d6 = _pad_edges(ei_rev_topic)

    c0, c1, c2, c4, c5, c6 = _sc_counts(
        [d0, d1, d2, d4, d5, d6],
        [n_p, n_p, n_a, n_a, n_f, n_p],
        core_of=[0, 0, 1, 1, 1, 1])
    c0, c1, c6 = c0[:n_p, 0:1], c1[:n_p, 0:1], c6[:n_p, 0:1]
    c2, c4, c5 = c2[:n_a, 0:1], c4[:n_a, 0:1], c5[:n_f, 0:1]

    y0h0, y0h1, y2h0, y2h1, y5h0, y5h1 = _mm_multi(
        x_paper, jnp.concatenate([W_l[0, 0], W_l[0, 2], W_l[0, 5]], axis=1),
        [64] * 6)
    y1h0, y1h1 = _mm_multi(x_author, W_l[0, 1], [64, 64])
    y4h0, y4h1 = _mm_multi(x_institution, W_l[0, 4], [64, 64])
    y6h0, y6h1 = _mm_multi(x_fos, W_l[0, 6], [64, 64])

    a0, a1, a6 = _sc_agg(n_p, [(y0h0, y0h1, s0, d0),
                               (y1h0, y1h1, s1, d1),
                               (y6h0, y6h1, s6, d6)])
    a2, a4 = _sc_agg(n_a, [(y2h0, y2h1, s2, d2),
                           (y4h0, y4h1, s4, d4)])
    (a5,) = _sc_agg(n_f, [(y5h0, y5h1, s5, d5)])

    h_p = _combine([_assemble(a0, n_p), _assemble(a1, n_p),
                    _assemble(a6, n_p)],
                   [c0, c1, c6], x_paper,
                   jnp.stack([W_r[0, 0], W_r[0, 1], W_r[0, 6]]),
                   jnp.stack([b_l[0, 0], b_l[0, 1], b_l[0, 6]]))
    h_a = _combine([_assemble(a2, n_a), _assemble(a4, n_a)],
                   [c2, c4], x_author,
                   jnp.stack([W_r[0, 2], W_r[0, 4]]),
                   jnp.stack([b_l[0, 2], b_l[0, 4]]))
    h_f = _combine([_assemble(a5, n_f)], [c5], x_fos,
                   jnp.stack([W_r[0, 5]]), jnp.stack([b_l[0, 5]]))

    z0h0, z0h1 = _mm_multi(h_p, W_l[1, 0], [64, 64])
    z1h0, z1h1 = _mm_multi(h_a, W_l[1, 1], [64, 64])
    z6h0, z6h1 = _mm_multi(h_f, W_l[1, 6], [64, 64])
    g0, g1, g6 = _sc_agg(n_p, [(z0h0, z0h1, s0, d0),
                               (z1h0, z1h1, s1, d1),
                               (z6h0, z6h1, s6, d6)])

    m_pad = _cdiv(out_dim, 128) * 128
    w_out_p = jnp.pad(W_out, ((0, 0), (0, m_pad - out_dim)))
    b_out_p = jnp.pad(b_out, (0, m_pad - out_dim))[None, :]
    logits = _combine(
        [_assemble(g0, n_p), _assemble(g1, n_p), _assemble(g6, n_p)],
        [c0, c1, c6], h_p,
        jnp.stack([W_r[1, 0], W_r[1, 1], W_r[1, 6]]),
        jnp.stack([b_l[1, 0], b_l[1, 1], b_l[1, 6]]),
        proj_w=w_out_p, proj_b=b_out_p)
    return logits[:, :out_dim]
